# baseline (speedup 1.0000x reference)
import functools

import jax
import jax.numpy as jnp
from jax import lax
from jax.experimental import pallas as pl
from jax.experimental.pallas import tpu as pltpu
from jax.experimental.pallas import tpu_sc as plsc

_N_SUITS = 4
_N_RANKS = 13
_N_VALS = _N_SUITS * _N_RANKS
_D = 64
_N_CARDS = 819200

_NC = 2
_NS = 16
_NW = _NC * _NS
_L = 16

_CHUNK = _N_CARDS // _NW
_UNROLL = 4
_NBUF = 2
_VECS = _CHUNK // _L


def _sc_body(cards_hbm, suit_hbm, rank_hbm, card_hbm, out_hbm,
             cards_v, suit_v, rank_v, card_v, partial_v,
             sem_a, sem_b, sem_t):
    wid = lax.axis_index("s") * _NC + lax.axis_index("c")
    base = wid * _CHUNK
    half = _CHUNK // 2

    cp_a = pltpu.async_copy(cards_hbm.at[pl.ds(base, half)],
                            cards_v.at[pl.ds(0, half)], sem_a)
    cp_b = pltpu.async_copy(cards_hbm.at[pl.ds(base + half, half)],
                            cards_v.at[pl.ds(half, half)], sem_b)
    cp_s = pltpu.async_copy(suit_hbm, suit_v, sem_t)
    cp_r = pltpu.async_copy(rank_hbm, rank_v, sem_t)
    cp_c = pltpu.async_copy(card_hbm, card_v, sem_t)

    def _hist(*count_bufs):
        lanes = lax.iota(jnp.int32, _L)
        ones = jnp.ones((_L,), jnp.float32)
        zeros = jnp.zeros((_L,), jnp.float32)

        @plsc.parallel_loop(0, _N_VALS)
        def _zero(c):
            for buf in count_bufs:
                buf[pl.ds(c * _L, _L)] = zeros

        def _half_hist(vec_base):
            @plsc.parallel_loop(0, _VECS // 2 // _UNROLL, unroll=4)
            def _hist_loop(i):
                start = vec_base + i * (_L * _UNROLL)
                for k in range(_UNROLL):
                    cv = cards_v[pl.ds(start + k * _L, _L)]
                    plsc.addupdate_scatter(count_bufs[k % _NBUF],
                                           [cv * _L + lanes], ones)

        cp_a.wait()
        _half_hist(0)
        cp_b.wait()
        _half_hist(half)
        cp_s.wait()
        cp_r.wait()
        cp_c.wait()

        def _fold(c, acc):
            lane_tot = count_bufs[0][pl.ds(c * _L, _L)]
            for buf in count_bufs[1:]:
                lane_tot = lane_tot + buf[pl.ds(c * _L, _L)]
            w = jnp.sum(lane_tot)
            s = c // _N_RANKS
            r = c - s * _N_RANKS
            new = []
            for j in range(_D // _L):
                sl = pl.ds(j * _L, _L)
                t = suit_v[s, sl] + rank_v[r, sl] + card_v[c, sl]
                new.append(acc[j] + w * t)
            return tuple(new)

        acc = lax.fori_loop(
            0, _N_VALS, _fold,
            tuple(jnp.zeros((_L,), jnp.float32) for _ in range(_D // _L)))
        for j in range(_D // _L):
            partial_v[pl.ds(j * _L, _L)] = acc[j]

    pl.run_scoped(_hist, *([pltpu.VMEM((_N_VALS * _L,), jnp.float32)] * _NBUF))
    pltpu.sync_copy(partial_v, out_hbm.at[wid])


@jax.jit
def kernel(cards, suit_embedding, rank_embedding, card_embedding):
    partials = pl.kernel(
        _sc_body,
        out_type=jax.ShapeDtypeStruct((_NW, _D), jnp.float32),
        mesh=plsc.VectorSubcoreMesh(core_axis_name="c", subcore_axis_name="s",
                                    num_cores=_NC, num_subcores=_NS),
        compiler_params=pltpu.CompilerParams(needs_layout_passes=False,
                                             skip_device_barrier=True),
        scratch_types=[
            pltpu.VMEM((_CHUNK,), jnp.int32),
            pltpu.VMEM((_N_SUITS, _D), jnp.float32),
            pltpu.VMEM((_N_RANKS, _D), jnp.float32),
            pltpu.VMEM((_N_VALS, _D), jnp.float32),
            pltpu.VMEM((_D,), jnp.float32),
            pltpu.SemaphoreType.DMA,
            pltpu.SemaphoreType.DMA,
            pltpu.SemaphoreType.DMA,
        ],
    )(cards, suit_embedding, rank_embedding, card_embedding)
    return jnp.sum(partials, axis=0)


# speedup vs baseline: 1.0026x; 1.0026x over previous
import functools

import jax
import jax.numpy as jnp
from jax import lax
from jax.experimental import pallas as pl
from jax.experimental.pallas import tpu as pltpu
from jax.experimental.pallas import tpu_sc as plsc

_N_SUITS = 4
_N_RANKS = 13
_N_VALS = _N_SUITS * _N_RANKS
_D = 64
_N_CARDS = 819200

_NC = 2
_NS = 16
_NW = _NC * _NS
_L = 16

_CHUNK = _N_CARDS // _NW
_UNROLL = 4
_NBUF = 2
_VECS = _CHUNK // _L


def _sc_body(cards_hbm, suit_hbm, rank_hbm, card_hbm, out_hbm,
             cards_v, suit_v, rank_v, card_v, partial_v,
             sem_a, sem_b, sem_t):
    wid = lax.axis_index("s") * _NC + lax.axis_index("c")
    base = wid * _CHUNK
    half = _CHUNK // 2

    cp_a = pltpu.async_copy(cards_hbm.at[pl.ds(base, half)],
                            cards_v.at[pl.ds(0, half)], sem_a)
    cp_b = pltpu.async_copy(cards_hbm.at[pl.ds(base + half, half)],
                            cards_v.at[pl.ds(half, half)], sem_b)
    cp_s = pltpu.async_copy(suit_hbm, suit_v, sem_t)
    cp_r = pltpu.async_copy(rank_hbm, rank_v, sem_t)
    cp_c = pltpu.async_copy(card_hbm, card_v, sem_t)

    def _hist(*count_bufs):
        lanes = lax.iota(jnp.int32, _L)
        ones = jnp.ones((_L,), jnp.float32)
        zeros = jnp.zeros((_L,), jnp.float32)

        @plsc.parallel_loop(0, _N_VALS)
        def _zero(c):
            for buf in count_bufs:
                buf[pl.ds(c * _L, _L)] = zeros

        def _half_hist(vec_base):
            @plsc.parallel_loop(0, _VECS // 2 // _UNROLL, unroll=2)
            def _hist_loop(i):
                start = vec_base + i * (_L * _UNROLL)
                for k in range(_UNROLL):
                    cv = cards_v[pl.ds(start + k * _L, _L)]
                    plsc.addupdate_scatter(count_bufs[k % _NBUF],
                                           [cv * _L + lanes], ones)

        cp_a.wait()
        _half_hist(0)
        cp_b.wait()
        _half_hist(half)
        cp_s.wait()
        cp_r.wait()
        cp_c.wait()

        nj = _D // _L
        suit_rows = [[suit_v[s, pl.ds(j * _L, _L)] for j in range(nj)]
                     for s in range(_N_SUITS)]

        def _fold(r, acc):
            rank_row = [rank_v[r, pl.ds(j * _L, _L)] for j in range(nj)]
            new = []
            for s in range(_N_SUITS):
                c = s * _N_RANKS + r
                lane_tot = count_bufs[0][pl.ds(c * _L, _L)]
                for buf in count_bufs[1:]:
                    lane_tot = lane_tot + buf[pl.ds(c * _L, _L)]
                w = jnp.sum(lane_tot)
                row = []
                for j in range(nj):
                    t = suit_rows[s][j] + rank_row[j] \
                        + card_v[c, pl.ds(j * _L, _L)]
                    row.append(acc[s * nj + j] + w * t)
                new.extend(row)
            return tuple(new)

        acc = lax.fori_loop(
            0, _N_RANKS, _fold,
            tuple(jnp.zeros((_L,), jnp.float32)
                  for _ in range(_N_SUITS * nj)))
        for j in range(nj):
            tot = acc[j]
            for s in range(1, _N_SUITS):
                tot = tot + acc[s * nj + j]
            partial_v[pl.ds(j * _L, _L)] = tot

    pl.run_scoped(_hist, *([pltpu.VMEM((_N_VALS * _L,), jnp.float32)] * _NBUF))
    pltpu.sync_copy(partial_v, out_hbm.at[wid])


@jax.jit
def kernel(cards, suit_embedding, rank_embedding, card_embedding):
    partials = pl.kernel(
        _sc_body,
        out_type=jax.ShapeDtypeStruct((_NW, _D), jnp.float32),
        mesh=plsc.VectorSubcoreMesh(core_axis_name="c", subcore_axis_name="s",
                                    num_cores=_NC, num_subcores=_NS),
        compiler_params=pltpu.CompilerParams(needs_layout_passes=False,
                                             skip_device_barrier=True),
        scratch_types=[
            pltpu.VMEM((_CHUNK,), jnp.int32),
            pltpu.VMEM((_N_SUITS, _D), jnp.float32),
            pltpu.VMEM((_N_RANKS, _D), jnp.float32),
            pltpu.VMEM((_N_VALS, _D), jnp.float32),
            pltpu.VMEM((_D,), jnp.float32),
            pltpu.SemaphoreType.DMA,
            pltpu.SemaphoreType.DMA,
            pltpu.SemaphoreType.DMA,
        ],
    )(cards, suit_embedding, rank_embedding, card_embedding)
    return jnp.sum(partials, axis=0)


# speedup vs baseline: 1.0048x; 1.0022x over previous
import jax
import jax.numpy as jnp
from jax import lax
from jax.experimental import pallas as pl
from jax.experimental.pallas import tpu as pltpu
from jax.experimental.pallas import tpu_sc as plsc

_N_SUITS = 4
_N_RANKS = 13
_N_VALS = _N_SUITS * _N_RANKS
_D = 64
_N_CARDS = 819200

_NC = 2
_NS = 16
_NW = _NC * _NS
_L = 16

_CHUNK = _N_CARDS // _NW
_UNROLL = 4
_NBUF = 2
_VECS = _CHUNK // _L


def _sc_body(cards_hbm, suit_hbm, rank_hbm, card_hbm, out_hbm,
             cards_v, suit_v, rank_v, card_v, partial_v,
             sem_a, sem_b, sem_t):
    wid = lax.axis_index("s") * _NC + lax.axis_index("c")
    base = wid * _CHUNK
    half = _CHUNK // 2

    cp_a = pltpu.async_copy(cards_hbm.at[pl.ds(base, half)],
                            cards_v.at[pl.ds(0, half)], sem_a)
    cp_b = pltpu.async_copy(cards_hbm.at[pl.ds(base + half, half)],
                            cards_v.at[pl.ds(half, half)], sem_b)
    cp_s = pltpu.async_copy(suit_hbm, suit_v, sem_t)
    cp_r = pltpu.async_copy(rank_hbm, rank_v, sem_t)
    cp_c = pltpu.async_copy(card_hbm, card_v, sem_t)

    def _hist(*count_bufs):
        lanes = lax.iota(jnp.int32, _L)
        ones = jnp.ones((_L,), jnp.float32)
        zeros = jnp.zeros((_L,), jnp.float32)

        @plsc.parallel_loop(0, _N_VALS)
        def _zero(c):
            for buf in count_bufs:
                buf[pl.ds(c * _L, _L)] = zeros

        def _half_hist(vec_base):
            @plsc.parallel_loop(0, _VECS // 2 // _UNROLL, unroll=2)
            def _hist_loop(i):
                start = vec_base + i * (_L * _UNROLL)
                for k in range(_UNROLL):
                    cv = cards_v[pl.ds(start + k * _L, _L)]
                    plsc.addupdate_scatter(count_bufs[k % _NBUF],
                                           [cv * _L + lanes], ones)

        cp_a.wait()
        _half_hist(0)
        cp_b.wait()
        _half_hist(half)
        cp_s.wait()
        cp_r.wait()
        cp_c.wait()

        nj = _D // _L
        suit_rows = [[suit_v[s, pl.ds(j * _L, _L)] for j in range(nj)]
                     for s in range(_N_SUITS)]

        def _fold(r, acc):
            rank_row = [rank_v[r, pl.ds(j * _L, _L)] for j in range(nj)]
            new = []
            for s in range(_N_SUITS):
                c = s * _N_RANKS + r
                lane_tot = count_bufs[0][pl.ds(c * _L, _L)]
                for buf in count_bufs[1:]:
                    lane_tot = lane_tot + buf[pl.ds(c * _L, _L)]
                w = jnp.sum(lane_tot)
                row = []
                for j in range(nj):
                    t = suit_rows[s][j] + rank_row[j] \
                        + card_v[c, pl.ds(j * _L, _L)]
                    row.append(acc[s * nj + j] + w * t)
                new.extend(row)
            return tuple(new)

        acc = lax.fori_loop(
            0, _N_RANKS, _fold,
            tuple(jnp.zeros((_L,), jnp.float32)
                  for _ in range(_N_SUITS * nj)))
        for j in range(nj):
            tot = acc[j]
            for s in range(1, _N_SUITS):
                tot = tot + acc[s * nj + j]
            partial_v[pl.ds(j * _L, _L)] = tot

    pl.run_scoped(_hist, *([pltpu.VMEM((_N_VALS * _L,), jnp.float32)] * _NBUF))
    pltpu.sync_copy(partial_v, out_hbm.at[wid])


@jax.jit
def kernel(cards, suit_embedding, rank_embedding, card_embedding):
    partials = pl.kernel(
        _sc_body,
        out_type=jax.ShapeDtypeStruct((_NW, _D), jnp.float32),
        mesh=plsc.VectorSubcoreMesh(core_axis_name="c", subcore_axis_name="s",
                                    num_cores=_NC, num_subcores=_NS),
        compiler_params=pltpu.CompilerParams(needs_layout_passes=False,
                                             skip_device_barrier=True),
        scratch_types=[
            pltpu.VMEM((_CHUNK,), jnp.int32),
            pltpu.VMEM((_N_SUITS, _D), jnp.float32),
            pltpu.VMEM((_N_RANKS, _D), jnp.float32),
            pltpu.VMEM((_N_VALS, _D), jnp.float32),
            pltpu.VMEM((_D,), jnp.float32),
            pltpu.SemaphoreType.DMA,
            pltpu.SemaphoreType.DMA,
            pltpu.SemaphoreType.DMA,
        ],
    )(cards, suit_embedding, rank_embedding, card_embedding)
    return jnp.sum(partials, axis=0)
